# baseline (speedup 1.0000x reference)
import functools
import jax
import jax.numpy as jnp
from jax import lax
from jax.experimental import pallas as pl
from jax.experimental.pallas import tpu as pltpu
from jax.experimental.pallas import tpu_sc as plsc

MAX_SEQ = 8192
SEQ = 512
NH = 8
HD = 128
NW = 32
RPW = MAX_SEQ // NW
CH = 32
NCHW = RPW // CH


def _sc_body(ipos, kv, vv, kc, vc, ko, vo, idx_v, buf0, buf1, sem):
    w = lax.axis_index("s") * 2 + lax.axis_index("c")
    base = w * RPW

    pltpu.sync_copy(ipos.at[pl.ds(0, 16)], idx_v)
    start = idx_v[...][0]

    bufs = (buf0, buf1)
    for which, (cache, val, out) in enumerate(((kc, kv, ko), (vc, vv, vo))):
        for t in range(NCHW):
            buf = bufs[(which * NCHW + t) % 2]
            c0 = base + t * CH
            full_in = jnp.logical_and(c0 >= start, c0 + CH <= start + SEQ)
            voff = jnp.maximum(c0 - start, 0)

            @pl.when(full_in)
            def _():
                pltpu.sync_copy(val.at[0, pl.ds(voff, CH)], buf)

            @pl.when(jnp.logical_not(full_in))
            def _():
                pltpu.sync_copy(cache.at[0, pl.ds(c0, CH)], buf)

            pltpu.sync_copy(buf, out.at[0, pl.ds(c0, CH)])


def kernel(input_pos, k_val, v_val, k_cache, v_cache):
    shp = k_cache.shape
    mesh = plsc.VectorSubcoreMesh(core_axis_name="c", subcore_axis_name="s")
    f = functools.partial(
        pl.kernel,
        mesh=mesh,
        out_type=[
            jax.ShapeDtypeStruct(shp, jnp.float32),
            jax.ShapeDtypeStruct(shp, jnp.float32),
        ],
        scratch_types=[
            pltpu.VMEM((16,), jnp.int32),
            pltpu.VMEM((CH, NH, HD), jnp.float32),
            pltpu.VMEM((CH, NH, HD), jnp.float32),
            pltpu.SemaphoreType.DMA,
        ],
    )(_sc_body)
    return tuple(f(input_pos, k_val, v_val, k_cache, v_cache))


# speedup vs baseline: 1.2549x; 1.2549x over previous
import functools
import jax
import jax.numpy as jnp
from jax import lax
from jax.experimental import pallas as pl
from jax.experimental.pallas import tpu as pltpu
from jax.experimental.pallas import tpu_sc as plsc

MAX_SEQ = 8192
SEQ = 512
NH = 8
HD = 128
R = 4096

NW = 32
RPW = R // NW
CH = 32

BLK = 512


def _sc_body(ipos, vv, vc, v1, idx_v, buf0, buf1):
    w = lax.axis_index("s") * 2 + lax.axis_index("c")
    base = w * RPW

    pltpu.sync_copy(ipos.at[pl.ds(0, 16)], idx_v)
    start = idx_v[...][0]

    bufs = (buf0, buf1)
    for t in range(RPW // CH):
        buf = bufs[t % 2]
        c0 = base + t * CH
        full_in = jnp.logical_and(c0 >= start, c0 + CH <= start + SEQ)
        voff = jnp.maximum(c0 - start, 0)

        @pl.when(full_in)
        def _():
            pltpu.sync_copy(vv.at[0, pl.ds(voff, CH)], buf)

        @pl.when(jnp.logical_not(full_in))
        def _():
            pltpu.sync_copy(vc.at[0, pl.ds(c0, CH)], buf)

        pltpu.sync_copy(buf, v1.at[0, pl.ds(c0, CH)])


def _ring(start, val, cache, out, row0, nch, buf, sem_in, sem_out, d, lag):
    def start_in(j):
        b = j % d
        c0 = row0 + j * BLK
        full_in = jnp.logical_and(c0 >= start, c0 + BLK <= start + SEQ)
        voff = jnp.clip(c0 - start, 0, SEQ - BLK) if BLK < SEQ else 0

        @pl.when(full_in)
        def _():
            pltpu.make_async_copy(
                val.at[0, pl.ds(voff, BLK)], buf.at[b], sem_in.at[b]).start()

        @pl.when(jnp.logical_not(full_in))
        def _():
            pltpu.make_async_copy(
                cache.at[0, pl.ds(c0, BLK)], buf.at[b], sem_in.at[b]).start()

    def wait_in(j):
        b = j % d
        pltpu.make_async_copy(
            cache.at[0, pl.ds(row0 + j * BLK, BLK)], buf.at[b],
            sem_in.at[b]).wait()

    def out_copy(j):
        b = j % d
        return pltpu.make_async_copy(
            buf.at[b], out.at[0, pl.ds(row0 + j * BLK, BLK)], sem_out.at[b])

    for j in range(min(d, nch)):
        start_in(j)
    for j in range(nch):
        wait_in(j)
        out_copy(j).start()
        if j >= lag and j - lag + d < nch:
            out_copy(j - lag).wait()
            start_in(j - lag + d)
    for j in range(max(0, nch - d), nch):
        out_copy(j).wait()


def _tc_k_body(s_ref, kv, kc, ko, buf, sem_in, sem_out, sem2):
    start = pl.multiple_of(s_ref[0], 8)
    _ring(start, kv, kc, ko, 0, MAX_SEQ // BLK, buf, sem_in, sem_out, 12, 6)

    @pl.when(jax.lax.rem(start, BLK) != 0)
    def _():
        cp = pltpu.make_async_copy(kv.at[0], ko.at[0, pl.ds(start, SEQ)], sem2)
        cp.start()
        cp.wait()


def _tc_vtail_body(s_ref, vv, vc, v1, vo, buf, sem_in, sem_out, sem2):
    start = pl.multiple_of(s_ref[0], 8)
    _ring(start, vv, vc, vo, R, (MAX_SEQ - R) // BLK, buf, sem_in, sem_out,
          8, 4)

    @pl.when(jax.lax.rem(start, BLK) != 0)
    def _():
        cp = pltpu.make_async_copy(vv.at[0], vo.at[0, pl.ds(start, SEQ)], sem2)
        cp.start()
        cp.wait()


def kernel(input_pos, k_val, v_val, k_cache, v_cache):
    shp = k_cache.shape
    start = jnp.clip(input_pos[0], 0, MAX_SEQ - SEQ).reshape(1).astype(jnp.int32)

    mesh = plsc.VectorSubcoreMesh(core_axis_name="c", subcore_axis_name="s")
    v1 = pl.kernel(
        _sc_body,
        mesh=mesh,
        out_type=jax.ShapeDtypeStruct(shp, jnp.float32),
        scratch_types=[
            pltpu.VMEM((16,), jnp.int32),
            pltpu.VMEM((CH, NH, HD), jnp.float32),
            pltpu.VMEM((CH, NH, HD), jnp.float32),
        ],
    )(input_pos, v_val, v_cache)

    hbm = pl.BlockSpec(memory_space=pltpu.MemorySpace.HBM)

    def tc_call(body, n_in, d, aliases):
        return pl.pallas_call(
            body,
            grid_spec=pltpu.PrefetchScalarGridSpec(
                num_scalar_prefetch=1,
                grid=(1,),
                in_specs=[hbm] * n_in,
                out_specs=hbm,
                scratch_shapes=[
                    pltpu.VMEM((d, BLK, NH, HD), jnp.float32),
                    pltpu.SemaphoreType.DMA((d,)),
                    pltpu.SemaphoreType.DMA((d,)),
                    pltpu.SemaphoreType.DMA,
                ],
            ),
            out_shape=jax.ShapeDtypeStruct(shp, jnp.float32),
            input_output_aliases=aliases,
        )

    ko = tc_call(_tc_k_body, 2, 12, {})(start, k_val, k_cache)
    vo = tc_call(_tc_vtail_body, 3, 8, {3: 0})(start, v_val, v_cache, v1)
    return (ko, vo)
